# baseline (speedup 1.0000x reference)
import functools

import jax
import jax.numpy as jnp
from jax import lax
from jax.experimental import pallas as pl
from jax.experimental.pallas import tpu as pltpu
from jax.experimental.pallas import tpu_sc as plsc

_VOCAB = 100000
_FACTOR = 128
_HIDDEN = 1024
_B = 4
_S = 2048
_NTOK = _B * _S

_NC = 2
_NS = 16
_NW = _NC * _NS
_PER_W = _NTOK // _NW
_CHUNK = 128
_CHUNKS = _PER_W // _CHUNK

_BS = 2048


_WPB = _S // _PER_W


def _sc_gather_body(idx_hbm, table_hbm, out_hbm, idx_v, rows_v, gsem, wsem):
    wid = lax.axis_index("s") * _NC + lax.axis_index("c")
    b = wid // _WPB
    off = (wid % _WPB) * _PER_W
    for j in range(_CHUNKS):
        pltpu.sync_copy(idx_hbm.at[b, pl.ds(off + j * _CHUNK, _CHUNK)], idx_v.at[j])
    gathers = [
        pltpu.async_copy(
            table_hbm.at[idx_v.at[j]],
            rows_v.at[pl.ds(j * _CHUNK, _CHUNK)],
            gsem,
        )
        for j in range(_CHUNKS)
    ]
    for cp in gathers:
        cp.wait()
    pltpu.async_copy(rows_v, out_hbm.at[pl.ds(wid * _PER_W, _PER_W)], wsem).wait()


@functools.cache
def _sc_gather():
    return pl.kernel(
        _sc_gather_body,
        mesh=plsc.VectorSubcoreMesh(core_axis_name="c", subcore_axis_name="s"),
        out_type=jax.ShapeDtypeStruct((_NTOK, _FACTOR), jnp.float32),
        scratch_types=[
            pltpu.VMEM((_CHUNKS, _CHUNK), jnp.int32),
            pltpu.VMEM((_PER_W, _FACTOR), jnp.float32),
            pltpu.SemaphoreType.DMA,
            pltpu.SemaphoreType.DMA,
        ],
    )


def _tc_body(g_ref, w2_ref, pos_ref, seg_ref, gam_ref, bet_ref, out_ref):
    seg = seg_ref[0, 0, :][:, None]
    onehot = (seg == lax.broadcasted_iota(jnp.int32, (1, 8), 1)).astype(jnp.float32)
    g_aug = jnp.concatenate([g_ref[0], onehot], axis=1)
    x = jnp.dot(g_aug, w2_ref[...], preferred_element_type=jnp.float32)
    x = x + pos_ref[...]
    mean = jnp.mean(x, axis=-1, keepdims=True)
    xc = x - mean
    var = jnp.mean(xc * xc, axis=-1, keepdims=True)
    xn = xc * lax.rsqrt(var + 1e-12)
    out_ref[0] = xn * gam_ref[...] + bet_ref[...]


def kernel(input_ids, segment_ids, word_table, W2, pos_table, type_table, gamma, beta):
    gathered = _sc_gather()(input_ids.astype(jnp.int32), word_table)

    nblk = _S // _BS
    g3 = gathered.reshape(_NTOK // _BS, _BS, _FACTOR)
    seg3 = segment_ids.astype(jnp.int32).reshape(_NTOK // _BS, 1, _BS)
    w2_aug = jnp.concatenate(
        [W2, type_table, jnp.zeros((5, _HIDDEN), jnp.float32)], axis=0
    )

    out = pl.pallas_call(
        _tc_body,
        grid=(nblk, _B),
        in_specs=[
            pl.BlockSpec((1, _BS, _FACTOR), lambda s, b: (b * nblk + s, 0, 0)),
            pl.BlockSpec((_FACTOR + 8, _HIDDEN), lambda s, b: (0, 0)),
            pl.BlockSpec((_BS, _HIDDEN), lambda s, b: (s, 0)),
            pl.BlockSpec((1, 1, _BS), lambda s, b: (b * nblk + s, 0, 0)),
            pl.BlockSpec((1, _HIDDEN), lambda s, b: (0, 0)),
            pl.BlockSpec((1, _HIDDEN), lambda s, b: (0, 0)),
        ],
        out_specs=pl.BlockSpec((1, _BS, _HIDDEN), lambda s, b: (b, s, 0)),
        out_shape=jax.ShapeDtypeStruct((_B, _S, _HIDDEN), jnp.float32),
        compiler_params=pltpu.CompilerParams(
            dimension_semantics=("parallel", "parallel"),
        ),
    )(g3, w2_aug, pos_table, seg3, gamma.reshape(1, _HIDDEN), beta.reshape(1, _HIDDEN))
    return out


# speedup vs baseline: 1.0113x; 1.0113x over previous
import functools

import jax
import jax.numpy as jnp
from jax import lax
from jax.experimental import pallas as pl
from jax.experimental.pallas import tpu as pltpu
from jax.experimental.pallas import tpu_sc as plsc

_VOCAB = 100000
_FACTOR = 128
_HIDDEN = 1024
_B = 4
_S = 2048
_NTOK = _B * _S

_NC = 2
_NS = 16
_NW = _NC * _NS
_PER_W = _NTOK // _NW
_CHUNK = 128
_CHUNKS = _PER_W // _CHUNK

_BS = 2048


_WPB = _S // _PER_W


def _sc_gather_body(idx_hbm, table_hbm, out_hbm, idx_v, rows_v, gsem, wsem):
    wid = lax.axis_index("s") * _NC + lax.axis_index("c")
    b = wid // _WPB
    off = (wid % _WPB) * _PER_W
    idx_copies = [
        pltpu.async_copy(
            idx_hbm.at[b, pl.ds(off + j * _CHUNK, _CHUNK)], idx_v.at[j], wsem
        )
        for j in range(_CHUNKS)
    ]
    for cp in idx_copies:
        cp.wait()
    gathers = [
        pltpu.async_copy(
            table_hbm.at[idx_v.at[j]],
            rows_v.at[pl.ds(j * _CHUNK, _CHUNK)],
            gsem,
        )
        for j in range(_CHUNKS)
    ]
    for cp in gathers:
        cp.wait()
    pltpu.async_copy(rows_v, out_hbm.at[pl.ds(wid * _PER_W, _PER_W)], wsem).wait()


@functools.cache
def _sc_gather():
    return pl.kernel(
        _sc_gather_body,
        mesh=plsc.VectorSubcoreMesh(core_axis_name="c", subcore_axis_name="s"),
        out_type=jax.ShapeDtypeStruct((_NTOK, _FACTOR), jnp.float32),
        scratch_types=[
            pltpu.VMEM((_CHUNKS, _CHUNK), jnp.int32),
            pltpu.VMEM((_PER_W, _FACTOR), jnp.float32),
            pltpu.SemaphoreType.DMA,
            pltpu.SemaphoreType.DMA,
        ],
    )


def _tc_body(g_ref, w2_ref, pos_ref, seg_ref, gam_ref, bet_ref, out_ref):
    seg = seg_ref[0, 0, :][:, None]
    onehot = (seg == lax.broadcasted_iota(jnp.int32, (1, 8), 1)).astype(jnp.float32)
    g_aug = jnp.concatenate([g_ref[0], onehot], axis=1)
    x = jnp.dot(g_aug, w2_ref[...], preferred_element_type=jnp.float32)
    x = x + pos_ref[...]
    mean = jnp.mean(x, axis=-1, keepdims=True)
    xc = x - mean
    var = jnp.mean(xc * xc, axis=-1, keepdims=True)
    xn = xc * lax.rsqrt(var + 1e-12)
    out_ref[0] = xn * gam_ref[...] + bet_ref[...]


def kernel(input_ids, segment_ids, word_table, W2, pos_table, type_table, gamma, beta):
    gathered = _sc_gather()(input_ids.astype(jnp.int32), word_table)

    nblk = _S // _BS
    g3 = gathered.reshape(_NTOK // _BS, _BS, _FACTOR)
    seg3 = segment_ids.astype(jnp.int32).reshape(_NTOK // _BS, 1, _BS)
    w2_aug = jnp.concatenate(
        [W2, type_table, jnp.zeros((5, _HIDDEN), jnp.float32)], axis=0
    )

    out = pl.pallas_call(
        _tc_body,
        grid=(nblk, _B),
        in_specs=[
            pl.BlockSpec((1, _BS, _FACTOR), lambda s, b: (b * nblk + s, 0, 0)),
            pl.BlockSpec((_FACTOR + 8, _HIDDEN), lambda s, b: (0, 0)),
            pl.BlockSpec((_BS, _HIDDEN), lambda s, b: (s, 0)),
            pl.BlockSpec((1, 1, _BS), lambda s, b: (b * nblk + s, 0, 0)),
            pl.BlockSpec((1, _HIDDEN), lambda s, b: (0, 0)),
            pl.BlockSpec((1, _HIDDEN), lambda s, b: (0, 0)),
        ],
        out_specs=pl.BlockSpec((1, _BS, _HIDDEN), lambda s, b: (b, s, 0)),
        out_shape=jax.ShapeDtypeStruct((_B, _S, _HIDDEN), jnp.float32),
        compiler_params=pltpu.CompilerParams(
            dimension_semantics=("parallel", "parallel"),
        ),
    )(g3, w2_aug, pos_table, seg3, gamma.reshape(1, _HIDDEN), beta.reshape(1, _HIDDEN))
    return out
